# baseline (speedup 1.0000x reference)
import functools

import jax
import jax.numpy as jnp
import numpy as np
from jax import lax
from jax.experimental import pallas as pl
from jax.experimental.pallas import tpu as pltpu
from jax.experimental.pallas import tpu_sc as plsc

N_NODE = 10000
IN_DIM = 128
NWORD = IN_DIM // 2
E = 320000
B_Q = 16384
REL_PAD = 10240

NC = 2
NS = 16
NW = NC * NS
EDGES_PER_TILE = E // NW
CHUNK = 40
NCHUNKS = EDGES_PER_TILE // CHUNK
QP_TILE = B_Q // NW
ROWS_A = 632
ROWS_B = N_NODE - 15 * ROWS_A
_IDX_OFFS = (0, 16, 24)

_SC_MESH = plsc.VectorSubcoreMesh(core_axis_name="c", subcore_axis_name="s")

_PERM = np.concatenate([
    np.concatenate([np.arange(16 * k, 16 * k + 16),
                    np.arange(64 + 16 * k, 64 + 16 * k + 16)])
    for k in range(4)])

_MASK_HI = np.int32(-65536)


def _pack_tc(x):
    u = lax.bitcast_convert_type(x.astype(jnp.bfloat16), jnp.uint16)
    w = (u[:, 64:].astype(jnp.uint32) << 16) | u[:, :64].astype(jnp.uint32)
    return lax.bitcast_convert_type(w, jnp.int32)


def _aqf_body(r_ref, wqr_ref, b_ref, aq_ref):
    aq_ref[...] = lax.dot_general(
        r_ref[...], wqr_ref[...], (((1,), (1,)), ((), ())),
        preferred_element_type=jnp.float32) + b_ref[...]


def _packs_body(h_ref, r_ref, ws_ref, wr_ref, t1_ref, t2_ref):
    h = h_ref[...]
    a = lax.dot_general(h, ws_ref[...], (((1,), (1,)), ((), ())),
                        preferred_element_type=jnp.float32)
    t1_ref[:, :NWORD] = _pack_tc(a)
    t1_ref[:, NWORD:] = _pack_tc(h)
    r = r_ref[...]
    a = lax.dot_general(r, wr_ref[...], (((1,), (1,)), ((), ())),
                        preferred_element_type=jnp.float32)
    t2_ref[:, :NWORD] = _pack_tc(a)
    t2_ref[:, NWORD:] = _pack_tc(r)


def _final_body(p_ref, wh_ref, out_ref):
    s = p_ref[0] + p_ref[1]
    out_ref[...] = lax.dot_general(
        s, wh_ref[...], (((1,), (1,)), ((), ())),
        preferred_element_type=jnp.float32)


def _sc_aq_body(aqf_hbm, qrel_hbm, out_hbm, idx_v, rows_v, sem):
    cid = lax.axis_index("c")
    sid = lax.axis_index("s")
    wid = sid * NC + cid
    base = wid * QP_TILE

    for k in range(QP_TILE // 128):
        pltpu.sync_copy(qrel_hbm.at[pl.ds(base + 128 * k, 128)], idx_v)
        pltpu.async_copy(aqf_hbm.at[idx_v], rows_v, sem).wait()
        pltpu.sync_copy(rows_v, out_hbm.at[pl.ds(base + 128 * k, 128)])


_sc_aq = functools.partial(
    pl.kernel,
    out_type=jax.ShapeDtypeStruct((B_Q, IN_DIM), jnp.float32),
    mesh=_SC_MESH,
    compiler_params=pltpu.CompilerParams(needs_layout_passes=False),
    scratch_types=[
        pltpu.VMEM((128,), jnp.int32),
        pltpu.VMEM((128, IN_DIM), jnp.float32),
        pltpu.SemaphoreType.DMA,
    ],
)(_sc_aq_body)


def _unpack(words):
    lo = plsc.bitcast(jnp.left_shift(words, 16), jnp.float32)
    hi = plsc.bitcast(jnp.bitwise_and(words, _MASK_HI), jnp.float32)
    return lo, hi


def _sc_edge_body(subc_hbm, relc_hbm, ridxc_hbm, objc_hbm,
                  t1_hbm, t2_hbm, t3_hbm, wv_hbm, out_hbm,
                  sub0, sub1, rel0, rel1, ridx0, ridx1,
                  obji0, obji1, obj0, obj1,
                  bs0, bs1, br0, br1, bq0, bq1, msg0, msg1,
                  wv_v, acc,
                  cs0, cs1, ss0, ss1,
                  g00, g01, g02, g10, g11, g12):
    cid = lax.axis_index("c")
    sid = lax.axis_index("s")
    wid = sid * NC + cid
    ebase = wid * EDGES_PER_TILE

    subv = [sub0, sub1]
    relv = [rel0, rel1]
    ridxv = [ridx0, ridx1]
    objiv = [obji0, obji1]
    objv = [obj0, obj1]
    bufs_s = [bs0, bs1]
    bufs_r = [br0, br1]
    bufs_q = [bq0, bq1]
    msgv = [msg0, msg1]
    csem = [cs0, cs1]
    ssem = [ss0, ss1]
    gsem = [[g00, g01, g02], [g10, g11, g12]]

    pltpu.sync_copy(wv_hbm, wv_v)

    w_regs = [wv_v[pl.ds(16 * j, 16)] for j in range(8)]
    wb_vec = wv_v[pl.ds(IN_DIM, 16)]

    def _idx_dma(ci, p):
        sl = pl.ds(ebase + ci * CHUNK, CHUNK)
        pltpu.async_copy(subc_hbm.at[sl], subv[p], csem[p])
        pltpu.async_copy(relc_hbm.at[sl], relv[p], csem[p])
        pltpu.async_copy(ridxc_hbm.at[sl], ridxv[p], csem[p])
        pltpu.async_copy(objc_hbm.at[sl], objiv[p], csem[p])

    def _idx_wait(ci, p):
        sl = pl.ds(ebase + ci * CHUNK, CHUNK)
        pltpu.make_async_copy(subc_hbm.at[sl], subv[p], csem[p]).wait()
        pltpu.make_async_copy(relc_hbm.at[sl], relv[p], csem[p]).wait()
        pltpu.make_async_copy(ridxc_hbm.at[sl], ridxv[p], csem[p]).wait()
        pltpu.make_async_copy(objc_hbm.at[sl], objiv[p], csem[p]).wait()

    def _gather_issue(p):
        pltpu.async_copy(t1_hbm.at[subv[p]], bufs_s[p], gsem[p][0])
        pltpu.async_copy(t2_hbm.at[relv[p]], bufs_r[p], gsem[p][1])
        pltpu.async_copy(t3_hbm.at[ridxv[p]], bufs_q[p], gsem[p][2])

    def _compute(p):
        b_s, b_r, b_q, msg = bufs_s[p], bufs_r[p], bufs_q[p], msgv[p]

        def _edge(e, icarry):
            accv = wb_vec
            for g in range(4):
                sl = pl.ds(16 * g, 16)
                a = (plsc.bitcast(b_s[e, sl], jnp.bfloat16)
                     + plsc.bitcast(b_r[e, sl], jnp.bfloat16))
                lo, hi = _unpack(plsc.bitcast(a, jnp.int32))
                tlo = jnp.maximum(lo + b_q[e, pl.ds(32 * g, 16)], 0.0)
                thi = jnp.maximum(hi + b_q[e, pl.ds(32 * g + 16, 16)], 0.0)
                accv = accv + tlo * w_regs[2 * g] + thi * w_regs[2 * g + 1]
            ssum = jnp.sum(accv)
            sv = jnp.full((16,), ssum, jnp.float32)
            av = 1.0 / (1.0 + jnp.exp(-sv))
            for g in range(4):
                sl = pl.ds(NWORD + 16 * g, 16)
                m = (plsc.bitcast(b_s[e, sl], jnp.bfloat16)
                     + plsc.bitcast(b_r[e, sl], jnp.bfloat16))
                lo, hi = _unpack(plsc.bitcast(m, jnp.int32))
                msg[e, pl.ds(32 * g, 16)] = lo * av
                msg[e, pl.ds(32 * g + 16, 16)] = hi * av
            return icarry

        lax.fori_loop(0, CHUNK, _edge, 0)

    def _obj_copy(p):
        for off in _IDX_OFFS:
            sl = pl.ds(off, 16)
            objv[p][sl] = objiv[p][sl]

    def _half(ci, p):
        q = 1 - p

        @pl.when(ci + 1 < NCHUNKS)
        def _():
            _idx_wait(ci + 1, q)

        @pl.when(ci >= 1)
        def _():
            pltpu.make_async_copy(
                msgv[q], acc.at[objv[q]], ssem[q]).wait()

        @pl.when(ci + 1 < NCHUNKS)
        def _():
            _obj_copy(q)
            _gather_issue(q)

        pltpu.make_async_copy(t1_hbm.at[subv[p]], bufs_s[p], gsem[p][0]).wait()
        pltpu.make_async_copy(t2_hbm.at[relv[p]], bufs_r[p], gsem[p][1]).wait()
        pltpu.make_async_copy(t3_hbm.at[ridxv[p]], bufs_q[p], gsem[p][2]).wait()

        @pl.when(ci + 2 < NCHUNKS)
        def _():
            _idx_dma(ci + 2, p)

        _compute(p)
        pltpu.async_copy(msgv[p], acc.at[objv[p]], ssem[p], add=True)

    _idx_dma(0, 0)
    zero16 = jnp.zeros((16,), jnp.float32)
    for i in range(CHUNK):
        for j in range(8):
            msg0[i, pl.ds(16 * j, 16)] = zero16
    row0 = sid * ROWS_A
    nz = jnp.where(sid < 15, ROWS_A // CHUNK, ROWS_B // CHUNK)

    def _zacc(z, carry):
        pltpu.sync_copy(msg0, acc.at[pl.ds(row0 + z * CHUNK, CHUNK)])
        return carry

    lax.fori_loop(0, nz, _zacc, 0)

    @pl.when(sid < 15)
    def _():
        pltpu.sync_copy(msg0.at[pl.ds(0, 32)],
                        acc.at[pl.ds(row0 + 15 * CHUNK, 32)])

    _idx_wait(0, 0)
    _obj_copy(0)
    _gather_issue(0)
    _idx_dma(1, 1)
    plsc.subcore_barrier()

    def _pair(i, carry):
        _half(2 * i, 0)
        _half(2 * i + 1, 1)
        return carry

    lax.fori_loop(0, NCHUNKS // 2, _pair, 0)
    pltpu.make_async_copy(msgv[1], acc.at[objv[1]], ssem[1]).wait()

    plsc.subcore_barrier()

    @pl.when(sid < 15)
    def _():
        pltpu.sync_copy(acc.at[pl.ds(sid * ROWS_A, ROWS_A)],
                        out_hbm.at[cid, pl.ds(sid * ROWS_A, ROWS_A)])

    @pl.when(sid == 15)
    def _():
        pltpu.sync_copy(acc.at[pl.ds(15 * ROWS_A, ROWS_B)],
                        out_hbm.at[cid, pl.ds(15 * ROWS_A, ROWS_B)])


_sc_edge = functools.partial(
    pl.kernel,
    out_type=jax.ShapeDtypeStruct((NC, N_NODE, IN_DIM), jnp.float32),
    mesh=_SC_MESH,
    compiler_params=pltpu.CompilerParams(needs_layout_passes=False),
    scratch_types=(
        [pltpu.VMEM((CHUNK,), jnp.int32)] * 10
        + [pltpu.VMEM((CHUNK, 2 * NWORD), jnp.int32)] * 4
        + [pltpu.VMEM((CHUNK, IN_DIM), jnp.float32)] * 2
        + [pltpu.VMEM((CHUNK, IN_DIM), jnp.float32)] * 2
        + [pltpu.VMEM((144,), jnp.float32)]
        + [pltpu.VMEM_SHARED((N_NODE, IN_DIM), jnp.float32)]
        + [pltpu.SemaphoreType.DMA] * 10
    ),
)(_sc_edge_body)


def kernel(q_sub, q_rel, hidden, edges, nodes, old_nodes_new_idx,
           rela_embed, Ws_attn, Wr_attn, Wqr_attn_w, Wqr_attn_b,
           w_alpha_w, w_alpha_b, W_h):
    sub_col = edges[:, 4]
    rel_col = edges[:, 2]
    ridx_col = edges[:, 0]
    obj_col = edges[:, 5]

    hid_pad = jnp.pad(hidden, ((0, REL_PAD - hidden.shape[0]), (0, 0)))
    rel_pad = jnp.pad(rela_embed, ((0, REL_PAD - rela_embed.shape[0]), (0, 0)))
    aqf = pl.pallas_call(
        _aqf_body,
        grid=(REL_PAD // 256,),
        in_specs=[
            pl.BlockSpec((256, IN_DIM), lambda i: (i, 0)),
            pl.BlockSpec((IN_DIM, IN_DIM), lambda i: (0, 0)),
            pl.BlockSpec((1, IN_DIM), lambda i: (0, 0)),
        ],
        out_specs=pl.BlockSpec((256, IN_DIM), lambda i: (i, 0)),
        out_shape=jax.ShapeDtypeStruct((REL_PAD, IN_DIM), jnp.float32),
    )(rel_pad, Wqr_attn_w[_PERM, :], Wqr_attn_b[_PERM].reshape(1, IN_DIM))

    t3 = _sc_aq(aqf, q_rel.astype(jnp.int32))

    t1, t2 = pl.pallas_call(
        _packs_body,
        grid=(REL_PAD // 256,),
        in_specs=[
            pl.BlockSpec((256, IN_DIM), lambda i: (i, 0)),
            pl.BlockSpec((256, IN_DIM), lambda i: (i, 0)),
            pl.BlockSpec((IN_DIM, IN_DIM), lambda i: (0, 0)),
            pl.BlockSpec((IN_DIM, IN_DIM), lambda i: (0, 0)),
        ],
        out_specs=[
            pl.BlockSpec((256, IN_DIM), lambda i: (i, 0)),
            pl.BlockSpec((256, IN_DIM), lambda i: (i, 0)),
        ],
        out_shape=[
            jax.ShapeDtypeStruct((REL_PAD, IN_DIM), jnp.int32),
            jax.ShapeDtypeStruct((REL_PAD, IN_DIM), jnp.int32),
        ],
    )(hid_pad, rel_pad, Ws_attn, Wr_attn)

    wv = jnp.concatenate([
        w_alpha_w.reshape(-1)[_PERM], w_alpha_b.reshape(-1),
        jnp.zeros((15,), jnp.float32)])

    partials = _sc_edge(sub_col, rel_col, ridx_col, obj_col, t1, t2, t3, wv)

    out = pl.pallas_call(
        _final_body,
        grid=(N_NODE // 400,),
        in_specs=[
            pl.BlockSpec((NC, 400, IN_DIM), lambda i: (0, i, 0)),
            pl.BlockSpec((IN_DIM, IN_DIM), lambda i: (0, 0)),
        ],
        out_specs=pl.BlockSpec((400, IN_DIM), lambda i: (i, 0)),
        out_shape=jax.ShapeDtypeStruct((N_NODE, IN_DIM), jnp.float32),
    )(partials, W_h[:, _PERM])
    return out


# speedup vs baseline: 1.0805x; 1.0805x over previous
import functools

import jax
import jax.numpy as jnp
import numpy as np
from jax import lax
from jax.experimental import pallas as pl
from jax.experimental.pallas import tpu as pltpu
from jax.experimental.pallas import tpu_sc as plsc

N_NODE = 10000
IN_DIM = 128
NWORD = IN_DIM // 2
E = 320000
B_Q = 16384
REL_PAD = 10240

NC = 2
NS = 16
NW = NC * NS
EDGES_PER_TILE = E // NW
CHUNK = 40
NCHUNKS = EDGES_PER_TILE // CHUNK
QP_TILE = B_Q // NW
ROWS_A = 632
ROWS_B = N_NODE - 15 * ROWS_A
_IDX_OFFS = (0, 16, 24)

_SC_MESH = plsc.VectorSubcoreMesh(core_axis_name="c", subcore_axis_name="s")

_PERM = np.concatenate([
    np.concatenate([np.arange(16 * k, 16 * k + 16),
                    np.arange(64 + 16 * k, 64 + 16 * k + 16)])
    for k in range(4)])

_MASK_HI = np.int32(-65536)


def _pack_tc(x):
    u = lax.bitcast_convert_type(x.astype(jnp.bfloat16), jnp.uint16)
    w = (u[:, 64:].astype(jnp.uint32) << 16) | u[:, :64].astype(jnp.uint32)
    return lax.bitcast_convert_type(w, jnp.int32)


def _tabs_body(h_ref, r_ref, ws_ref, wr_ref, wqr_ref, b_ref,
               t1_ref, t2_ref, aq_ref):
    h = h_ref[...]
    a = lax.dot_general(h, ws_ref[...], (((1,), (1,)), ((), ())),
                        preferred_element_type=jnp.float32)
    t1_ref[:, :NWORD] = _pack_tc(a)
    t1_ref[:, NWORD:] = _pack_tc(h)
    r = r_ref[...]
    a = lax.dot_general(r, wr_ref[...], (((1,), (1,)), ((), ())),
                        preferred_element_type=jnp.float32)
    t2_ref[:, :NWORD] = _pack_tc(a)
    t2_ref[:, NWORD:] = _pack_tc(r)
    aq_ref[...] = lax.dot_general(
        r, wqr_ref[...], (((1,), (1,)), ((), ())),
        preferred_element_type=jnp.float32) + b_ref[...]


def _final_body(p_ref, wh_ref, out_ref):
    s = p_ref[0] + p_ref[1]
    out_ref[...] = lax.dot_general(
        s, wh_ref[...], (((1,), (1,)), ((), ())),
        preferred_element_type=jnp.float32)


def _sc_aq_body(aqf_hbm, qrel_hbm, out_hbm, idx_v, rows_v, sem):
    cid = lax.axis_index("c")
    sid = lax.axis_index("s")
    wid = sid * NC + cid
    base = wid * QP_TILE

    for k in range(QP_TILE // 128):
        pltpu.sync_copy(qrel_hbm.at[pl.ds(base + 128 * k, 128)], idx_v)
        pltpu.async_copy(aqf_hbm.at[idx_v], rows_v, sem).wait()
        pltpu.sync_copy(rows_v, out_hbm.at[pl.ds(base + 128 * k, 128)])


_sc_aq = functools.partial(
    pl.kernel,
    out_type=jax.ShapeDtypeStruct((B_Q, IN_DIM), jnp.float32),
    mesh=_SC_MESH,
    compiler_params=pltpu.CompilerParams(needs_layout_passes=False),
    scratch_types=[
        pltpu.VMEM((128,), jnp.int32),
        pltpu.VMEM((128, IN_DIM), jnp.float32),
        pltpu.SemaphoreType.DMA,
    ],
)(_sc_aq_body)


def _unpack(words):
    lo = plsc.bitcast(jnp.left_shift(words, 16), jnp.float32)
    hi = plsc.bitcast(jnp.bitwise_and(words, _MASK_HI), jnp.float32)
    return lo, hi


def _sc_edge_body(subc_hbm, relc_hbm, ridxc_hbm, objc_hbm,
                  t1_hbm, t2_hbm, t3_hbm, wv_hbm, out_hbm,
                  sub0, sub1, rel0, rel1, ridx0, ridx1,
                  obji0, obji1, obj0, obj1,
                  bs0, bs1, br0, br1, bq0, bq1, msg0, msg1,
                  wv_v, acc,
                  cs0, cs1, ss0, ss1,
                  g00, g01, g02, g10, g11, g12):
    cid = lax.axis_index("c")
    sid = lax.axis_index("s")
    wid = sid * NC + cid
    ebase = wid * EDGES_PER_TILE

    subv = [sub0, sub1]
    relv = [rel0, rel1]
    ridxv = [ridx0, ridx1]
    objiv = [obji0, obji1]
    objv = [obj0, obj1]
    bufs_s = [bs0, bs1]
    bufs_r = [br0, br1]
    bufs_q = [bq0, bq1]
    msgv = [msg0, msg1]
    csem = [cs0, cs1]
    ssem = [ss0, ss1]
    gsem = [[g00, g01, g02], [g10, g11, g12]]

    pltpu.sync_copy(wv_hbm, wv_v)

    w_regs = [wv_v[pl.ds(16 * j, 16)] for j in range(8)]
    wb_vec = wv_v[pl.ds(IN_DIM, 16)]

    def _idx_dma(ci, p):
        sl = pl.ds(ebase + ci * CHUNK, CHUNK)
        pltpu.async_copy(subc_hbm.at[sl], subv[p], csem[p])
        pltpu.async_copy(relc_hbm.at[sl], relv[p], csem[p])
        pltpu.async_copy(ridxc_hbm.at[sl], ridxv[p], csem[p])
        pltpu.async_copy(objc_hbm.at[sl], objiv[p], csem[p])

    def _idx_wait(ci, p):
        sl = pl.ds(ebase + ci * CHUNK, CHUNK)
        pltpu.make_async_copy(subc_hbm.at[sl], subv[p], csem[p]).wait()
        pltpu.make_async_copy(relc_hbm.at[sl], relv[p], csem[p]).wait()
        pltpu.make_async_copy(ridxc_hbm.at[sl], ridxv[p], csem[p]).wait()
        pltpu.make_async_copy(objc_hbm.at[sl], objiv[p], csem[p]).wait()

    def _gather_issue(p):
        pltpu.async_copy(t1_hbm.at[subv[p]], bufs_s[p], gsem[p][0])
        pltpu.async_copy(t2_hbm.at[relv[p]], bufs_r[p], gsem[p][1])
        pltpu.async_copy(t3_hbm.at[ridxv[p]], bufs_q[p], gsem[p][2])

    def _compute(p):
        b_s, b_r, b_q, msg = bufs_s[p], bufs_r[p], bufs_q[p], msgv[p]

        def _edge(e, icarry):
            accv = wb_vec
            for g in range(4):
                sl = pl.ds(16 * g, 16)
                a = (plsc.bitcast(b_s[e, sl], jnp.bfloat16)
                     + plsc.bitcast(b_r[e, sl], jnp.bfloat16))
                lo, hi = _unpack(plsc.bitcast(a, jnp.int32))
                tlo = jnp.maximum(lo + b_q[e, pl.ds(32 * g, 16)], 0.0)
                thi = jnp.maximum(hi + b_q[e, pl.ds(32 * g + 16, 16)], 0.0)
                accv = accv + tlo * w_regs[2 * g] + thi * w_regs[2 * g + 1]
            ssum = jnp.sum(accv)
            sv = jnp.full((16,), ssum, jnp.float32)
            av = 1.0 / (1.0 + jnp.exp(-sv))
            for g in range(4):
                sl = pl.ds(NWORD + 16 * g, 16)
                m = (plsc.bitcast(b_s[e, sl], jnp.bfloat16)
                     + plsc.bitcast(b_r[e, sl], jnp.bfloat16))
                lo, hi = _unpack(plsc.bitcast(m, jnp.int32))
                msg[e, pl.ds(32 * g, 16)] = lo * av
                msg[e, pl.ds(32 * g + 16, 16)] = hi * av
            return icarry

        lax.fori_loop(0, CHUNK, _edge, 0)

    def _obj_copy(p):
        for off in _IDX_OFFS:
            sl = pl.ds(off, 16)
            objv[p][sl] = objiv[p][sl]

    def _half(ci, p):
        q = 1 - p

        @pl.when(ci + 1 < NCHUNKS)
        def _():
            _idx_wait(ci + 1, q)

        @pl.when(ci >= 1)
        def _():
            pltpu.make_async_copy(
                msgv[q], acc.at[objv[q]], ssem[q]).wait()

        @pl.when(ci + 1 < NCHUNKS)
        def _():
            _obj_copy(q)
            _gather_issue(q)

        pltpu.make_async_copy(t1_hbm.at[subv[p]], bufs_s[p], gsem[p][0]).wait()
        pltpu.make_async_copy(t2_hbm.at[relv[p]], bufs_r[p], gsem[p][1]).wait()
        pltpu.make_async_copy(t3_hbm.at[ridxv[p]], bufs_q[p], gsem[p][2]).wait()

        @pl.when(ci + 2 < NCHUNKS)
        def _():
            _idx_dma(ci + 2, p)

        _compute(p)
        pltpu.async_copy(msgv[p], acc.at[objv[p]], ssem[p], add=True)

    _idx_dma(0, 0)
    zero16 = jnp.zeros((16,), jnp.float32)
    for i in range(CHUNK):
        for j in range(8):
            msg0[i, pl.ds(16 * j, 16)] = zero16
    row0 = sid * ROWS_A
    nz = jnp.where(sid < 15, ROWS_A // CHUNK, ROWS_B // CHUNK)

    def _zacc(z, carry):
        pltpu.sync_copy(msg0, acc.at[pl.ds(row0 + z * CHUNK, CHUNK)])
        return carry

    lax.fori_loop(0, nz, _zacc, 0)

    @pl.when(sid < 15)
    def _():
        pltpu.sync_copy(msg0.at[pl.ds(0, 32)],
                        acc.at[pl.ds(row0 + 15 * CHUNK, 32)])

    _idx_wait(0, 0)
    _obj_copy(0)
    _gather_issue(0)
    _idx_dma(1, 1)
    plsc.subcore_barrier()

    def _pair(i, carry):
        _half(2 * i, 0)
        _half(2 * i + 1, 1)
        return carry

    lax.fori_loop(0, NCHUNKS // 2, _pair, 0)
    pltpu.make_async_copy(msgv[1], acc.at[objv[1]], ssem[1]).wait()

    plsc.subcore_barrier()

    @pl.when(sid < 15)
    def _():
        pltpu.sync_copy(acc.at[pl.ds(sid * ROWS_A, ROWS_A)],
                        out_hbm.at[cid, pl.ds(sid * ROWS_A, ROWS_A)])

    @pl.when(sid == 15)
    def _():
        pltpu.sync_copy(acc.at[pl.ds(15 * ROWS_A, ROWS_B)],
                        out_hbm.at[cid, pl.ds(15 * ROWS_A, ROWS_B)])


_sc_edge = functools.partial(
    pl.kernel,
    out_type=jax.ShapeDtypeStruct((NC, N_NODE, IN_DIM), jnp.float32),
    mesh=_SC_MESH,
    compiler_params=pltpu.CompilerParams(needs_layout_passes=False),
    scratch_types=(
        [pltpu.VMEM((CHUNK,), jnp.int32)] * 10
        + [pltpu.VMEM((CHUNK, 2 * NWORD), jnp.int32)] * 4
        + [pltpu.VMEM((CHUNK, IN_DIM), jnp.float32)] * 2
        + [pltpu.VMEM((CHUNK, IN_DIM), jnp.float32)] * 2
        + [pltpu.VMEM((144,), jnp.float32)]
        + [pltpu.VMEM_SHARED((N_NODE, IN_DIM), jnp.float32)]
        + [pltpu.SemaphoreType.DMA] * 10
    ),
)(_sc_edge_body)


def kernel(q_sub, q_rel, hidden, edges, nodes, old_nodes_new_idx,
           rela_embed, Ws_attn, Wr_attn, Wqr_attn_w, Wqr_attn_b,
           w_alpha_w, w_alpha_b, W_h):
    sub_col = edges[:, 4]
    rel_col = edges[:, 2]
    ridx_col = edges[:, 0]
    obj_col = edges[:, 5]

    n_rel = rela_embed.shape[0]
    t1, t2, aqf = pl.pallas_call(
        _tabs_body,
        grid=(REL_PAD // 256,),
        in_specs=[
            pl.BlockSpec((256, IN_DIM), lambda i: (i, 0)),
            pl.BlockSpec((256, IN_DIM), lambda i: (i, 0)),
            pl.BlockSpec((IN_DIM, IN_DIM), lambda i: (0, 0)),
            pl.BlockSpec((IN_DIM, IN_DIM), lambda i: (0, 0)),
            pl.BlockSpec((IN_DIM, IN_DIM), lambda i: (0, 0)),
            pl.BlockSpec((1, IN_DIM), lambda i: (0, 0)),
        ],
        out_specs=[
            pl.BlockSpec((256, IN_DIM), lambda i: (i, 0)),
            pl.BlockSpec((256, IN_DIM), lambda i: (i, 0)),
            pl.BlockSpec((256, IN_DIM), lambda i: (i, 0)),
        ],
        out_shape=[
            jax.ShapeDtypeStruct((N_NODE, IN_DIM), jnp.int32),
            jax.ShapeDtypeStruct((n_rel, IN_DIM), jnp.int32),
            jax.ShapeDtypeStruct((n_rel, IN_DIM), jnp.float32),
        ],
    )(hidden, rela_embed, Ws_attn, Wr_attn,
      Wqr_attn_w[_PERM, :], Wqr_attn_b[_PERM].reshape(1, IN_DIM))

    t3 = _sc_aq(aqf, q_rel.astype(jnp.int32))

    wv = jnp.concatenate([
        w_alpha_w.reshape(-1)[_PERM], w_alpha_b.reshape(-1),
        jnp.zeros((15,), jnp.float32)])

    partials = _sc_edge(sub_col, rel_col, ridx_col, obj_col, t1, t2, t3, wv)

    out = pl.pallas_call(
        _final_body,
        grid=(N_NODE // 400,),
        in_specs=[
            pl.BlockSpec((NC, 400, IN_DIM), lambda i: (0, i, 0)),
            pl.BlockSpec((IN_DIM, IN_DIM), lambda i: (0, 0)),
        ],
        out_specs=pl.BlockSpec((400, IN_DIM), lambda i: (i, 0)),
        out_shape=jax.ShapeDtypeStruct((N_NODE, IN_DIM), jnp.float32),
    )(partials, W_h[:, _PERM])
    return out
